# baseline (speedup 1.0000x reference)
import functools

import jax
import jax.numpy as jnp
from jax import lax
from jax.experimental import pallas as pl
from jax.experimental.pallas import tpu as pltpu
from jax.experimental.pallas import tpu_sc as plsc

_INFO = plsc.get_sparse_core_info()
_NC = _INFO.num_cores
_NS = _INFO.num_subcores
_NW = _NC * _NS

_B = 1024
_L = 200
_D = 64
_ROWS = _B * _L
_RPW = _ROWS // _NW
_M = 80
_IDX_ROWS = _RPW // _M
_CHUNK = 2 * _L
_DMAS = _CHUNK // _M
_NCHUNK = _RPW // _CHUNK


def _make_sc_call():
    mesh = plsc.VectorSubcoreMesh(core_axis_name="c", subcore_axis_name="s")

    @functools.partial(
        pl.kernel,
        mesh=mesh,
        out_type=jax.ShapeDtypeStruct((_NW * _NCHUNK, _CHUNK, _D), jnp.float32),
        compiler_params=pltpu.CompilerParams(use_tc_tiling_on_sc=False),
        scratch_types=[
            pltpu.VMEM((_IDX_ROWS, _M), jnp.int32),
            pltpu.VMEM((_CHUNK, _D), jnp.float32),
            pltpu.VMEM((_L, _D), jnp.float32),
            pltpu.SemaphoreType.DMA,
        ],
    )
    def sc_kernel(x3_hbm, tok_hbm, pos_hbm, out_hbm, idx_v, buf, pos_v, sem):
        wid = lax.axis_index("s") * _NC + lax.axis_index("c")
        pltpu.sync_copy(x3_hbm.at[wid], idx_v)
        pltpu.sync_copy(pos_hbm, pos_v)

        def add_pos(p, _):
            for k in range(_D // 16):
                sl = pl.ds(16 * k, 16)
                pr = pos_v[p, sl]
                plsc.addupdate(buf.at[p, sl], pr)
                plsc.addupdate(buf.at[p + _L, sl], pr)
            return 0

        for c in range(_NCHUNK):
            handles = [
                pltpu.async_copy(
                    tok_hbm.at[idx_v.at[_DMAS * c + j]],
                    buf.at[pl.ds(_M * j, _M)],
                    sem,
                )
                for j in range(_DMAS)
            ]
            for h in handles:
                h.wait()
            lax.fori_loop(0, _L, add_pos, 0)
            pltpu.sync_copy(buf, out_hbm.at[wid * _NCHUNK + c])

    return sc_kernel


_sc_call = _make_sc_call()


def kernel(x, token_table, pos_table):
    b, l = x.shape
    v, d = token_table.shape
    x3 = x.astype(jnp.int32).reshape(_NW, _IDX_ROWS, _M)
    out = _sc_call(x3, token_table, pos_table)
    return out.reshape(b, l, d)


# speedup vs baseline: 1.0012x; 1.0012x over previous
import functools

import jax
import jax.numpy as jnp
from jax import lax
from jax.experimental import pallas as pl
from jax.experimental.pallas import tpu as pltpu
from jax.experimental.pallas import tpu_sc as plsc

_INFO = plsc.get_sparse_core_info()
_NC = _INFO.num_cores
_NS = _INFO.num_subcores
_NW = _NC * _NS

_B = 1024
_L = 200
_D = 64
_SPW = _B // _NW
_M = 40
_G = _L // _M
_NCHUNK = _SPW // 2


def _make_sc_call():
    mesh = plsc.VectorSubcoreMesh(core_axis_name="c", subcore_axis_name="s")

    @functools.partial(
        pl.kernel,
        mesh=mesh,
        out_type=jax.ShapeDtypeStruct((_B, _L, _D), jnp.float32),
        compiler_params=pltpu.CompilerParams(use_tc_tiling_on_sc=False),
        scratch_types=[
            pltpu.VMEM((_SPW, _L), jnp.int32),
            pltpu.VMEM((2, _L, _D), jnp.float32),
            pltpu.VMEM((_L, _D), jnp.float32),
            pltpu.SemaphoreType.DMA,
        ],
    )
    def sc_kernel(x_hbm, tok_hbm, pos_hbm, out_hbm, idx_v, buf, pos_v, sem):
        wid = lax.axis_index("s") * _NC + lax.axis_index("c")
        b_base = wid * _SPW
        pltpu.sync_copy(x_hbm.at[pl.ds(b_base, _SPW)], idx_v)
        pltpu.sync_copy(pos_hbm, pos_v)

        def add_pos(p, _):
            for k in range(_D // 16):
                sl = pl.ds(16 * k, 16)
                pr = pos_v[p, sl]
                plsc.addupdate(buf.at[0, p, sl], pr)
                plsc.addupdate(buf.at[1, p, sl], pr)
            return 0

        for c in range(_NCHUNK):
            handles = [
                pltpu.async_copy(
                    tok_hbm.at[idx_v.at[2 * c + s, pl.ds(_M * k, _M)]],
                    buf.at[s, pl.ds(_M * k, _M)],
                    sem,
                )
                for s in range(2)
                for k in range(_G)
            ]
            for h in handles:
                h.wait()
            lax.fori_loop(0, _L, add_pos, 0)
            pltpu.sync_copy(buf, out_hbm.at[pl.ds(b_base + 2 * c, 2)])

    return sc_kernel


_sc_call = _make_sc_call()


def kernel(x, token_table, pos_table):
    return _sc_call(x.astype(jnp.int32), token_table, pos_table)


# speedup vs baseline: 1.5566x; 1.5548x over previous
import functools

import jax
import jax.numpy as jnp
from jax import lax
from jax.experimental import pallas as pl
from jax.experimental.pallas import tpu as pltpu
from jax.experimental.pallas import tpu_sc as plsc
from jax.experimental.layout import Layout, Format, with_layout_constraint

_INFO = plsc.get_sparse_core_info()
_NC = _INFO.num_cores
_NS = _INFO.num_subcores
_NW = _NC * _NS

_B = 1024
_L = 200
_D = 64
_V = 1000000
_SPW = _B // _NW
_M = 40
_G = _L // _M
_NCHUNK = _SPW // 2
_IPW = _SPW * _L
_CH = 2 * _L


def _make_sc_call():
    mesh = plsc.VectorSubcoreMesh(core_axis_name="c", subcore_axis_name="s")

    @functools.partial(
        pl.kernel,
        mesh=mesh,
        out_type=jax.ShapeDtypeStruct((_B * _L, _D), jnp.float32),
        compiler_params=pltpu.CompilerParams(use_tc_tiling_on_sc=False),
        scratch_types=[
            pltpu.VMEM((_IPW,), jnp.int32),
            pltpu.VMEM((_CH, _D), jnp.float32),
            pltpu.VMEM((_L, _D), jnp.float32),
            pltpu.SemaphoreType.DMA,
        ],
    )
    def sc_kernel(x_hbm, tok_hbm, pos_hbm, out_hbm, idx_v, buf, pos_v, sem):
        wid = lax.axis_index("s") * _NC + lax.axis_index("c")
        b_base = wid * _SPW
        for s in range(_SPW):
            pltpu.sync_copy(x_hbm.at[b_base + s], idx_v.at[pl.ds(s * _L, _L)])
        pltpu.sync_copy(pos_hbm, pos_v)

        def add_pos(p, _):
            for k in range(4):
                sl = pl.ds(16 * k, 16)
                pr = pos_v[p, sl]
                plsc.addupdate(buf.at[p, sl], pr)
                plsc.addupdate(buf.at[p + _L, sl], pr)
            return 0

        for c in range(_NCHUNK):
            handles = [
                pltpu.async_copy(
                    tok_hbm.at[idx_v.at[pl.ds(c * _CH + _M * g, _M)]],
                    buf.at[pl.ds(_M * g, _M)],
                    sem,
                )
                for g in range(2 * _G)
            ]
            for h in handles:
                h.wait()
            lax.fori_loop(0, _L, add_pos, 0)
            pltpu.sync_copy(
                buf, out_hbm.at[pl.ds((b_base + 2 * c) * _L, _CH)])

    return sc_kernel


_sc_call = _make_sc_call()

def kernel(x, token_table, pos_table):
    lin2 = Layout(major_to_minor=(0, 1), tiling=((8,),))
    lin3 = Layout(major_to_minor=(0, 1, 2), tiling=((8,),))
    tok_lin = with_layout_constraint(token_table, lin2)
    out2 = _sc_call(x.astype(jnp.int32), tok_lin, pos_table)
    out3 = out2.reshape(_B, _L, _D)
    return with_layout_constraint(out3, lin3)
